# Initial kernel scaffold; baseline (speedup 1.0000x reference)
import jax
import jax.numpy as jnp
from jax.experimental import pallas as pl


def kernel(hidden_states, Wr, Wg_s, Wu_s, Wd_s, Wg_e, Wu_e, Wd_e):
    raise NotImplementedError("write your pallas kernel here")


# speedup vs baseline: 1.0316x; 1.0316x over previous
import functools
import jax
import jax.numpy as jnp
from jax import lax
from jax.experimental import pallas as pl
from jax.experimental.pallas import tpu as pltpu

B, S, D, H, E, K = 2, 2048, 1024, 512, 8, 2
EP = 128
M = 256


def _silu(x):
    return x * jax.nn.sigmoid(x)


def _moe_kernel(x_ref, wrp_ref, wgs_ref, wus_ref, wds_ref,
                wge_ref, wue_ref, wde_ref, out_ref, wtop_ref):
    e = pl.program_id(1)
    xt = x_ref[...]

    @pl.when(e == 0)
    def _first():
        gate = lax.dot_general(xt, wgs_ref[...], (((1,), (1,)), ((), ())),
                               preferred_element_type=jnp.float32)
        up = lax.dot_general(xt, wus_ref[...], (((1,), (1,)), ((), ())),
                             preferred_element_type=jnp.float32)
        act = _silu(gate) * up
        shared = lax.dot_general(act, wds_ref[...], (((1,), (1,)), ((), ())),
                                 preferred_element_type=jnp.float32)
        out_ref[...] = shared

        logits = lax.dot_general(xt, wrp_ref[...], (((1,), (1,)), ((), ())),
                                 preferred_element_type=jnp.float32)
        lane = lax.broadcasted_iota(jnp.int32, (M, EP), 1)
        valid = lane < E
        z = jnp.where(valid, logits, -jnp.inf)
        zmax = jnp.max(z, axis=1, keepdims=True)
        ex = jnp.exp(z - zmax)
        p = ex / jnp.sum(ex, axis=1, keepdims=True)
        mx1 = jnp.max(p, axis=1, keepdims=True)
        i1 = jnp.min(jnp.where(p >= mx1, lane, EP), axis=1, keepdims=True)
        p2 = jnp.where(lane == i1, -1.0, p)
        mx2 = jnp.max(p2, axis=1, keepdims=True)
        i2 = jnp.min(jnp.where(p2 >= mx2, lane, EP), axis=1, keepdims=True)
        sel = (lane == i1) | (lane == i2)
        wtop_ref[...] = jnp.where(sel, p, 0.0) / (mx1 + mx2)

    lane2 = lax.broadcasted_iota(jnp.int32, (M, EP), 1)
    w_e = jnp.sum(jnp.where(lane2 == e, wtop_ref[...], 0.0),
                  axis=1, keepdims=True)
    gate = lax.dot_general(xt, wge_ref[0], (((1,), (1,)), ((), ())),
                           preferred_element_type=jnp.float32)
    up = lax.dot_general(xt, wue_ref[0], (((1,), (1,)), ((), ())),
                         preferred_element_type=jnp.float32)
    act = _silu(gate) * up
    eo = lax.dot_general(act, wde_ref[0], (((1,), (1,)), ((), ())),
                         preferred_element_type=jnp.float32)
    out_ref[...] += w_e * eo


def kernel(hidden_states, Wr, Wg_s, Wu_s, Wd_s, Wg_e, Wu_e, Wd_e):
    b, s, d = hidden_states.shape
    T = b * s
    x = hidden_states.reshape(T, d)
    Wrp = jnp.zeros((EP, d), jnp.float32).at[:E].set(Wr)

    grid = (T // M, E)
    out = pl.pallas_call(
        _moe_kernel,
        grid=grid,
        in_specs=[
            pl.BlockSpec((M, D), lambda t, e: (t, 0)),
            pl.BlockSpec((EP, D), lambda t, e: (0, 0)),
            pl.BlockSpec((H, D), lambda t, e: (0, 0)),
            pl.BlockSpec((H, D), lambda t, e: (0, 0)),
            pl.BlockSpec((D, H), lambda t, e: (0, 0)),
            pl.BlockSpec((1, H, D), lambda t, e: (e, 0, 0)),
            pl.BlockSpec((1, H, D), lambda t, e: (e, 0, 0)),
            pl.BlockSpec((1, D, H), lambda t, e: (e, 0, 0)),
        ],
        out_specs=pl.BlockSpec((M, D), lambda t, e: (t, 0)),
        out_shape=jax.ShapeDtypeStruct((T, D), jnp.float32),
        scratch_shapes=[pltpu.VMEM((M, EP), jnp.float32)],
        compiler_params=pltpu.CompilerParams(
            dimension_semantics=("arbitrary", "arbitrary")),
    )(x, Wrp, Wg_s, Wu_s, Wd_s, Wg_e, Wu_e, Wd_e)
    return out.reshape(b, s, d)
